# baseline (speedup 1.0000x reference)
import functools
import math

import jax
import jax.numpy as jnp
from jax import lax
from jax.experimental import pallas as pl
from jax.experimental.pallas import tpu as pltpu
from jax.experimental.pallas import tpu_sc as plsc

D = 32
L = 16
CH = 128
SCALE = math.sqrt(D)


def _body(n_ch, w_hbm, i_hbm, out_hbm, idx_v, buf, gsem, ssem):
  nc = 2
  wid = lax.axis_index("s") * nc + lax.axis_index("c")

  pltpu.sync_copy(i_hbm.at[pl.ds(wid * n_ch, n_ch)], idx_v)

  def chunk(j, carry):
    pltpu.async_copy(w_hbm.at[idx_v.at[j]], buf, gsem).wait()

    for i in range(CH):
      buf[i, pl.ds(0, L)] = buf[i, pl.ds(0, L)] * SCALE
      buf[i, pl.ds(L, L)] = buf[i, pl.ds(L, L)] * SCALE

    row0 = (wid * n_ch + j) * CH
    pltpu.async_copy(buf, out_hbm.at[pl.ds(row0, CH)], ssem).wait()
    return carry

  lax.fori_loop(0, n_ch, chunk, 0, unroll=False)


def kernel(x, weight):
  B0, S = x.shape
  B = B0 * S
  nw = 32
  n_ch = B // (nw * CH)
  assert B % (nw * CH) == 0

  idx = x.reshape(B // CH, CH).astype(jnp.int32)

  mesh = plsc.VectorSubcoreMesh(core_axis_name="c", subcore_axis_name="s")
  k = pl.kernel(
      functools.partial(_body, n_ch),
      out_type=jax.ShapeDtypeStruct((B, D), jnp.float32),
      mesh=mesh,
      scratch_types=[
          pltpu.VMEM((n_ch, CH), jnp.int32),
          pltpu.VMEM((CH, D), jnp.float32),
          pltpu.SemaphoreType.DMA,
          pltpu.SemaphoreType.DMA,
      ],
      compiler_params=pltpu.CompilerParams(use_tc_tiling_on_sc=False),
  )
  out = k(weight, idx)
  return out.reshape(B0, S, D)
